# baseline (speedup 1.0000x reference)
import functools

import jax
import jax.numpy as jnp
from jax import lax
from jax.experimental import pallas as pl
from jax.experimental.pallas import tpu as pltpu
from jax.experimental.pallas import tpu_sc as plsc

NC, NS, L = 2, 16, 16
NW = NC * NS
CHUNK = 128
NBUF = 3
NRING = 6
POOL_ROWS = 272
DUMP = 256

BLK = 8192
N1_BLOCKS = 7


def _sc_segsum_body(feat_hbm, batch_hbm, out_hbm, rows_v, idx_v, zb_v, tmp_v,
                    pool_sh, sem0, sem1, sem2, sem3, sem4, sem5,
                    ssem0, ssem1, ssem2, ssem3, ssem4, ssem5, *,
                    row0, n_rows, nchunks, per_w):
    c = lax.axis_index("c")
    s = lax.axis_index("s")
    w = s * NC + c
    sems = (sem0, sem1, sem2, sem3, sem4, sem5)
    ssems = (ssem0, ssem1, ssem2, ssem3, ssem4, ssem5)

    for i in range(17):
        for j in range(8):
            zb_v[i, pl.ds(j * L, L)] = jnp.zeros((L,), jnp.float32)
    pltpu.sync_copy(zb_v, pool_sh.at[pl.ds(s * 17, 17)])
    plsc.subcore_barrier()

    def base_of(g):
        return row0 + jnp.minimum(g * CHUNK, (n_rows - row0) - CHUNK)

    def start(t, b):
        g = t * NW + w

        @pl.when(g < nchunks)
        def _():
            be = base_of(g)
            pltpu.async_copy(feat_hbm.at[pl.ds(be, CHUNK)], rows_v.at[b],
                             sems[b])
            pltpu.async_copy(batch_hbm.at[pl.ds(be, CHUNK)], idx_v.at[b],
                             sems[b])

    def wait_scatter(t, b):
        g = t * NW + w

        @pl.when((t >= 0) & (g < nchunks))
        def _():
            pltpu.make_async_copy(rows_v.at[b], pool_sh.at[idx_v.at[b]],
                                  ssems[b]).wait()

    def finish(t, b):
        g = t * NW + w

        @pl.when(g < nchunks)
        def _():
            base = row0 + g * CHUNK
            be = base_of(g)
            pltpu.make_async_copy(feat_hbm.at[pl.ds(be, CHUNK)],
                                  rows_v.at[b], sems[b]).wait()
            pltpu.make_async_copy(batch_hbm.at[pl.ds(be, CHUNK)],
                                  idx_v.at[b], sems[b]).wait()
            lane = lax.broadcasted_iota(jnp.int32, (L,), 0)
            for j in range(CHUNK // L):
                rid = be + j * L + lane
                v = idx_v[b, pl.ds(j * L, L)]
                idx_v[b, pl.ds(j * L, L)] = jnp.where(rid >= base, v, DUMP)
            pltpu.async_copy(rows_v.at[b], pool_sh.at[idx_v.at[b]], ssems[b],
                             add=True)

    for t0 in range(NBUF):
        start(t0, t0 % NRING)

    ngroups = (per_w + NRING - 1) // NRING

    def gbody(tg, carry):
        for k in range(NRING):
            t = tg * NRING + k
            finish(t, k)
            wait_scatter(t - NBUF, (k - NBUF) % NRING)
            start(t + NBUF, (k + NBUF) % NRING)
        return carry

    lax.fori_loop(0, ngroups, gbody, 0)
    last = ngroups * NRING
    for t in range(last - NBUF, last):
        wait_scatter(t, t % NRING)
    plsc.subcore_barrier()

    pltpu.sync_copy(pool_sh.at[pl.ds(s * L, L)], tmp_v)
    pltpu.sync_copy(tmp_v, out_hbm.at[c, pl.ds(s * L, L)])


def _sc_segsum(feat, batch, row0):
    n, d = feat.shape
    nchunks = (n - row0 + CHUNK - 1) // CHUNK
    per_w = (nchunks + NW - 1) // NW
    mesh = plsc.VectorSubcoreMesh(core_axis_name="c", subcore_axis_name="s",
                                  num_cores=NC, num_subcores=NS)
    f = pl.kernel(
        functools.partial(_sc_segsum_body, row0=row0, n_rows=n,
                          nchunks=nchunks, per_w=per_w),
        out_type=jax.ShapeDtypeStruct((NC, NS * L, d), jnp.float32),
        mesh=mesh,
        scratch_types=[
            pltpu.VMEM((NRING, CHUNK, d), jnp.float32),
            pltpu.VMEM((NRING, CHUNK), jnp.int32),
            pltpu.VMEM((17, d), jnp.float32),
            pltpu.VMEM((L, d), jnp.float32),
            pltpu.VMEM_SHARED((POOL_ROWS, d), jnp.float32),
        ] + [pltpu.SemaphoreType.DMA] * (2 * NRING),
    )
    return f(feat, batch)


def _tc_pool_body(feat_ref, batch_ref, out_ref, f16_ref, *, num_graphs):
    i = pl.program_id(0)

    @pl.when(i == 0)
    def _init():
        out_ref[...] = jnp.zeros_like(out_ref)

    bvec = batch_ref[0, 0, :]
    gids = lax.broadcasted_iota(jnp.int32, (num_graphs, BLK), 0)
    onehot_t = jnp.where(gids == bvec[None, :], 1.0, 0.0)
    out_ref[...] += jnp.dot(onehot_t, feat_ref[...],
                            preferred_element_type=jnp.float32)
    f16_ref[...] = feat_ref[...].astype(jnp.bfloat16)


def _tc_bc_body(feat_ref, f16_ref, batch_ref, psc_ref, ptc_ref, vn_ref,
                w_ref, b_ref, out_ref, vnout_ref, vn_scr, *, num_graphs):
    i = pl.program_id(0)

    @pl.when(i == 0)
    def _fc():
        pool = psc_ref[0] + psc_ref[1] + ptc_ref[...]
        vn_tmp = pool + vn_ref[...]
        vn_o = jnp.maximum(
            jnp.dot(vn_tmp, w_ref[...], preferred_element_type=jnp.float32)
            + b_ref[...], 0.0) + vn_ref[...]
        vn_scr[...] = vn_o
        vnout_ref[...] = vn_o

    bvec = batch_ref[0, 0, :]
    gids = lax.broadcasted_iota(jnp.int32, (BLK, num_graphs), 1)
    onehot = jnp.where(gids == bvec[:, None], 1.0, 0.0)
    vnb = jnp.dot(onehot, vn_scr[...], preferred_element_type=jnp.float32)

    @pl.when(i < N1_BLOCKS)
    def _cached():
        out_ref[...] = f16_ref[...].astype(jnp.float32) + vnb

    @pl.when(i >= N1_BLOCKS)
    def _hbm():
        out_ref[...] = feat_ref[...] + vnb


def kernel(feat, vn_feat, W, b, batch):
    n, d = feat.shape
    num_graphs = vn_feat.shape[0]
    n1 = N1_BLOCKS * BLK

    num_blocks = (n + BLK - 1) // BLK
    pad = num_blocks * BLK - n
    batch_r = jnp.pad(batch, (0, pad)).reshape(num_blocks, 1, BLK)

    p_sc = _sc_segsum(feat, batch, n1)

    p_tc, feat16 = pl.pallas_call(
        functools.partial(_tc_pool_body, num_graphs=num_graphs),
        grid=(N1_BLOCKS,),
        in_specs=[
            pl.BlockSpec((BLK, d), lambda i: (i, 0)),
            pl.BlockSpec((1, 1, BLK), lambda i: (i, 0, 0)),
        ],
        out_specs=[
            pl.BlockSpec((num_graphs, d), lambda i: (0, 0)),
            pl.BlockSpec((BLK, d), lambda i: (i, 0)),
        ],
        out_shape=(
            jax.ShapeDtypeStruct((num_graphs, d), jnp.float32),
            jax.ShapeDtypeStruct((n1, d), jnp.bfloat16),
        ),
        compiler_params=pltpu.CompilerParams(
            dimension_semantics=("arbitrary",),
        ),
    )(feat, batch_r)

    feat_out, vn_out = pl.pallas_call(
        functools.partial(_tc_bc_body, num_graphs=num_graphs),
        grid=(num_blocks,),
        in_specs=[
            pl.BlockSpec((BLK, d), lambda i: (jnp.maximum(i, N1_BLOCKS), 0)),
            pl.BlockSpec((BLK, d),
                         lambda i: (jnp.minimum(i, N1_BLOCKS - 1), 0)),
            pl.BlockSpec((1, 1, BLK), lambda i: (i, 0, 0)),
            pl.BlockSpec((NC, num_graphs, d), lambda i: (0, 0, 0)),
            pl.BlockSpec((num_graphs, d), lambda i: (0, 0)),
            pl.BlockSpec((num_graphs, d), lambda i: (0, 0)),
            pl.BlockSpec((d, d), lambda i: (0, 0)),
            pl.BlockSpec((1, d), lambda i: (0, 0)),
        ],
        out_specs=[
            pl.BlockSpec((BLK, d), lambda i: (i, 0)),
            pl.BlockSpec((num_graphs, d), lambda i: (0, 0)),
        ],
        scratch_shapes=[
            pltpu.VMEM((num_graphs, d), jnp.float32),
        ],
        out_shape=(
            jax.ShapeDtypeStruct((n, d), jnp.float32),
            jax.ShapeDtypeStruct((num_graphs, d), jnp.float32),
        ),
        compiler_params=pltpu.CompilerParams(
            dimension_semantics=("arbitrary",),
        ),
    )(feat, feat16, batch_r, p_sc, p_tc, vn_feat, W, b.reshape(1, d))
    return (feat_out, vn_out)


# speedup vs baseline: 1.0706x; 1.0706x over previous
import functools

import jax
import jax.numpy as jnp
from jax import lax
from jax.experimental import pallas as pl
from jax.experimental.pallas import tpu as pltpu
from jax.experimental.pallas import tpu_sc as plsc

NC, NS, L = 2, 16, 16
NW = NC * NS
CHUNK = 128
NBUF = 3
NRING = 6
POOL_ROWS = 272
DUMP = 256

BLK = 16384
N1_BLOCKS = 4


def _sc_segsum_body(feat_hbm, batch_hbm, out_hbm, rows_v, idx_v, zb_v, tmp_v,
                    pool_sh, sem0, sem1, sem2, sem3, sem4, sem5,
                    ssem0, ssem1, ssem2, ssem3, ssem4, ssem5, *,
                    row0, n_rows, nchunks, per_w):
    c = lax.axis_index("c")
    s = lax.axis_index("s")
    w = s * NC + c
    sems = (sem0, sem1, sem2, sem3, sem4, sem5)
    ssems = (ssem0, ssem1, ssem2, ssem3, ssem4, ssem5)

    def base_of(g):
        return row0 + jnp.minimum(g * CHUNK, (n_rows - row0) - CHUNK)

    def start(t, b):
        g = t * NW + w

        @pl.when(g < nchunks)
        def _():
            be = base_of(g)
            pltpu.async_copy(feat_hbm.at[pl.ds(be, CHUNK)], rows_v.at[b],
                             sems[b])
            pltpu.async_copy(batch_hbm.at[pl.ds(be, CHUNK)], idx_v.at[b],
                             sems[b])

    def wait_scatter(t, b):
        g = t * NW + w

        @pl.when((t >= 0) & (g < nchunks))
        def _():
            pltpu.make_async_copy(rows_v.at[b], pool_sh.at[idx_v.at[b]],
                                  ssems[b]).wait()

    def finish(t, b):
        g = t * NW + w

        @pl.when(g < nchunks)
        def _():
            base = row0 + g * CHUNK
            be = base_of(g)
            pltpu.make_async_copy(feat_hbm.at[pl.ds(be, CHUNK)],
                                  rows_v.at[b], sems[b]).wait()
            pltpu.make_async_copy(batch_hbm.at[pl.ds(be, CHUNK)],
                                  idx_v.at[b], sems[b]).wait()
            lane = lax.broadcasted_iota(jnp.int32, (L,), 0)
            for j in range(CHUNK // L):
                rid = be + j * L + lane
                v = idx_v[b, pl.ds(j * L, L)]
                idx_v[b, pl.ds(j * L, L)] = jnp.where(rid >= base, v, DUMP)
            pltpu.async_copy(rows_v.at[b], pool_sh.at[idx_v.at[b]], ssems[b],
                             add=True)

    for t0 in range(NBUF):
        start(t0, t0 % NRING)

    for i in range(17):
        for j in range(8):
            zb_v[i, pl.ds(j * L, L)] = jnp.zeros((L,), jnp.float32)
    pltpu.sync_copy(zb_v, pool_sh.at[pl.ds(s * 17, 17)])
    plsc.subcore_barrier()

    ngroups = (per_w + NRING - 1) // NRING

    def gbody(tg, carry):
        for k in range(NRING):
            t = tg * NRING + k
            finish(t, k)
            wait_scatter(t - NBUF, (k - NBUF) % NRING)
            start(t + NBUF, (k + NBUF) % NRING)
        return carry

    lax.fori_loop(0, ngroups, gbody, 0)
    last = ngroups * NRING
    for t in range(last - NBUF, last):
        wait_scatter(t, t % NRING)
    plsc.subcore_barrier()

    pltpu.sync_copy(pool_sh.at[pl.ds(s * L, L)], tmp_v)
    pltpu.sync_copy(tmp_v, out_hbm.at[c, pl.ds(s * L, L)])


def _sc_segsum(feat, batch, row0):
    n, d = feat.shape
    nchunks = (n - row0 + CHUNK - 1) // CHUNK
    per_w = (nchunks + NW - 1) // NW
    mesh = plsc.VectorSubcoreMesh(core_axis_name="c", subcore_axis_name="s",
                                  num_cores=NC, num_subcores=NS)
    f = pl.kernel(
        functools.partial(_sc_segsum_body, row0=row0, n_rows=n,
                          nchunks=nchunks, per_w=per_w),
        out_type=jax.ShapeDtypeStruct((NC, NS * L, d), jnp.float32),
        mesh=mesh,
        scratch_types=[
            pltpu.VMEM((NRING, CHUNK, d), jnp.float32),
            pltpu.VMEM((NRING, CHUNK), jnp.int32),
            pltpu.VMEM((17, d), jnp.float32),
            pltpu.VMEM((L, d), jnp.float32),
            pltpu.VMEM_SHARED((POOL_ROWS, d), jnp.float32),
        ] + [pltpu.SemaphoreType.DMA] * (2 * NRING),
    )
    return f(feat, batch)


def _tc_pool_body(feat_ref, batch_ref, out_ref, *, num_graphs):
    i = pl.program_id(0)

    @pl.when(i == 0)
    def _init():
        out_ref[...] = jnp.zeros_like(out_ref)

    bvec = batch_ref[0, 0, :]
    gids = lax.broadcasted_iota(jnp.int32, (num_graphs, BLK), 0)
    onehot_t = jnp.where(gids == bvec[None, :], 1.0, 0.0)
    out_ref[...] += jnp.dot(onehot_t, feat_ref[...],
                            preferred_element_type=jnp.float32)


def _tc_bc_body(feat_ref, batch_ref, psc_ref, ptc_ref, vn_ref,
                w_ref, b_ref, out_ref, vnout_ref, vn_scr, *, num_graphs):
    i = pl.program_id(0)

    @pl.when(i == 0)
    def _fc():
        pool = psc_ref[0] + psc_ref[1] + ptc_ref[...]
        vn_tmp = pool + vn_ref[...]
        vn_o = jnp.maximum(
            jnp.dot(vn_tmp, w_ref[...], preferred_element_type=jnp.float32)
            + b_ref[...], 0.0) + vn_ref[...]
        vn_scr[...] = vn_o
        vnout_ref[...] = vn_o

    bvec = batch_ref[0, 0, :]
    gids = lax.broadcasted_iota(jnp.int32, (BLK, num_graphs), 1)
    onehot = jnp.where(gids == bvec[:, None], 1.0, 0.0)
    out_ref[...] = feat_ref[...] + jnp.dot(
        onehot, vn_scr[...], preferred_element_type=jnp.float32)


def kernel(feat, vn_feat, W, b, batch):
    n, d = feat.shape
    num_graphs = vn_feat.shape[0]
    n1 = N1_BLOCKS * BLK

    num_blocks = (n + BLK - 1) // BLK
    pad = num_blocks * BLK - n
    batch_r = jnp.pad(batch, (0, pad)).reshape(num_blocks, 1, BLK)

    p_sc = _sc_segsum(feat, batch, n1)

    p_tc = pl.pallas_call(
        functools.partial(_tc_pool_body, num_graphs=num_graphs),
        grid=(N1_BLOCKS,),
        in_specs=[
            pl.BlockSpec((BLK, d), lambda i: (i, 0)),
            pl.BlockSpec((1, 1, BLK), lambda i: (i, 0, 0)),
        ],
        out_specs=pl.BlockSpec((num_graphs, d), lambda i: (0, 0)),
        out_shape=jax.ShapeDtypeStruct((num_graphs, d), jnp.float32),
        compiler_params=pltpu.CompilerParams(
            dimension_semantics=("arbitrary",),
        ),
    )(feat, batch_r)

    feat_out, vn_out = pl.pallas_call(
        functools.partial(_tc_bc_body, num_graphs=num_graphs),
        grid=(num_blocks,),
        in_specs=[
            pl.BlockSpec((BLK, d), lambda i: (i, 0)),
            pl.BlockSpec((1, 1, BLK), lambda i: (i, 0, 0)),
            pl.BlockSpec((NC, num_graphs, d), lambda i: (0, 0, 0)),
            pl.BlockSpec((num_graphs, d), lambda i: (0, 0)),
            pl.BlockSpec((num_graphs, d), lambda i: (0, 0)),
            pl.BlockSpec((d, d), lambda i: (0, 0)),
            pl.BlockSpec((1, d), lambda i: (0, 0)),
        ],
        out_specs=[
            pl.BlockSpec((BLK, d), lambda i: (i, 0)),
            pl.BlockSpec((num_graphs, d), lambda i: (0, 0)),
        ],
        scratch_shapes=[
            pltpu.VMEM((num_graphs, d), jnp.float32),
        ],
        out_shape=(
            jax.ShapeDtypeStruct((n, d), jnp.float32),
            jax.ShapeDtypeStruct((num_graphs, d), jnp.float32),
        ),
        compiler_params=pltpu.CompilerParams(
            dimension_semantics=("arbitrary",),
        ),
    )(feat, batch_r, p_sc, p_tc, vn_feat, W, b.reshape(1, d))
    return (feat_out, vn_out)
